# baseline (speedup 1.0000x reference)
import functools

import jax
import jax.numpy as jnp
from jax import lax
from jax.experimental import pallas as pl
from jax.experimental.pallas import tpu as pltpu
from jax.experimental.pallas import tpu_sc as plsc

_NC = 2
_NS = 16
_NW = _NC * _NS
_CH = 128


def _make_sc_segsum(F, Np, k0, k1):
    rpt = Np // _NS
    kmax = max(k0, k1)
    mesh = plsc.VectorSubcoreMesh(core_axis_name="c", subcore_axis_name="s")

    @functools.partial(
        pl.kernel,
        out_type=jax.ShapeDtypeStruct((_NC * Np, F), jnp.float32),
        mesh=mesh,
        scratch_types=[
            pltpu.VMEM((kmax, _CH), jnp.int32),
            pltpu.VMEM((kmax, _CH), jnp.int32),
            pltpu.VMEM((_CH, F), jnp.float32),
            pltpu.VMEM((_CH, F), jnp.float32),
            pltpu.VMEM((rpt, F), jnp.float32),
            pltpu.VMEM_SHARED((Np, F), jnp.float32),
            pltpu.SemaphoreType.DMA,
            pltpu.SemaphoreType.DMA,
        ],
        compiler_params=pltpu.CompilerParams(use_tc_tiling_on_sc=False),
    )
    def k(table_hbm, src_hbm, dst_hbm, zeros_hbm, out_hbm,
          sidx_v, didx_v, rows0, rows1, buf_v, acc_sh, sem0, sem1):
        cid = lax.axis_index("c")
        sid = lax.axis_index("s")
        kc = jnp.where(cid == 0, k0, k1)
        row0 = cid * _NS * k0 + sid * kc

        with jax.named_scope("idx_preload"):
            pltpu.sync_copy(src_hbm.at[pl.ds(row0, kmax)], sidx_v)
            pltpu.sync_copy(dst_hbm.at[pl.ds(row0, kmax)], didx_v)

        with jax.named_scope("acc_init"):
            pltpu.sync_copy(zeros_hbm, buf_v)
            pltpu.sync_copy(buf_v, acc_sh.at[pl.ds(sid * rpt, rpt)])
            plsc.subcore_barrier()

        kc_half = kc // 2
        pltpu.async_copy(table_hbm.at[sidx_v.at[0]], rows0, sem0)

        def body(i):
            @pl.when(i < kc_half)
            def _():
                j0 = 2 * i
                pltpu.make_async_copy(table_hbm.at[sidx_v.at[j0]], rows0, sem0
                                      ).wait()
                pltpu.async_copy(table_hbm.at[sidx_v.at[j0 + 1]], rows1, sem1)
                pltpu.sync_copy(rows0, acc_sh.at[didx_v.at[j0]], add=True)
                pltpu.make_async_copy(table_hbm.at[sidx_v.at[j0 + 1]], rows1,
                                      sem1).wait()

                @pl.when(i < kc_half - 1)
                def _():
                    pltpu.async_copy(table_hbm.at[sidx_v.at[j0 + 2]], rows0,
                                     sem0)

                pltpu.sync_copy(rows1, acc_sh.at[didx_v.at[j0 + 1]], add=True)

        with jax.named_scope("edge_loop"):
            pl.loop(0, kmax // 2)(body)
            plsc.subcore_barrier()

        with jax.named_scope("writeout"):
            pltpu.sync_copy(acc_sh.at[pl.ds(sid * rpt, rpt)], buf_v)
            pltpu.sync_copy(buf_v,
                            out_hbm.at[pl.ds(cid * Np + sid * rpt, rpt)])

    return k


def _tc1(x, W1l, W1r):
    N, D = x.shape
    H = W1l.shape[0]
    F1 = H + 16

    def body(x_ref, wl_ref, wr_ref, ya_ref, r_ref):
        xb = x_ref[...]
        y = lax.dot_general(xb, wl_ref[...], (((1,), (1,)), ((), ())),
                            preferred_element_type=jnp.float32)
        ones_col = (lax.broadcasted_iota(jnp.int32, (N, 16), 1) == 0
                    ).astype(jnp.float32)
        ya_ref[...] = jnp.concatenate([y, ones_col], axis=1)
        r_ref[...] = lax.dot_general(xb, wr_ref[...], (((1,), (1,)), ((), ())),
                                     preferred_element_type=jnp.float32)

    return pl.pallas_call(
        body,
        out_shape=[jax.ShapeDtypeStruct((N, F1), jnp.float32),
                   jax.ShapeDtypeStruct((N, H), jnp.float32)],
    )(x, W1l, W1r)


def _tc2(part1, r1, b1l, W2l, W2r, b2l, N, Np):
    H = r1.shape[1]
    O = W2l.shape[0]

    def body(p_ref, r1_ref, b1l_ref, wl_ref, wr_ref, b2l_ref,
             y2_ref, r2_ref, cnt_ref):
        p = p_ref[:N] + p_ref[Np:Np + N]
        cnt = jnp.maximum(p[:, H:H + 1], 1.0)
        h = jnp.maximum(p[:, :H] / cnt + b1l_ref[...] + r1_ref[...], 0.0)
        y2_ref[...] = lax.dot_general(h, wl_ref[...], (((1,), (1,)), ((), ())),
                                      preferred_element_type=jnp.float32)
        r2_ref[...] = lax.dot_general(h, wr_ref[...], (((1,), (1,)), ((), ())),
                                      preferred_element_type=jnp.float32) \
            + b2l_ref[...]
        cnt_ref[...] = jnp.broadcast_to(cnt, (N, O))

    return pl.pallas_call(
        body,
        out_shape=[jax.ShapeDtypeStruct((N, O), jnp.float32),
                   jax.ShapeDtypeStruct((N, O), jnp.float32),
                   jax.ShapeDtypeStruct((N, O), jnp.float32)],
    )(part1, r1, b1l, W2l, W2r, b2l)


def _tc3(part2, cnt, r2f, N, Np):
    O = cnt.shape[1]

    def body(p_ref, cnt_ref, r2_ref, o_ref):
        o_ref[...] = (p_ref[:N] + p_ref[Np:Np + N]) / cnt_ref[...] + r2_ref[...]

    return pl.pallas_call(
        body,
        out_shape=jax.ShapeDtypeStruct((N, O), jnp.float32),
    )(part2, cnt, r2f)


def _split(total, t1, t0):
    k0 = int(round(total * t1 / (t0 + t1) / 2)) * 2
    return min(max(k0, 2), total - 2)


def kernel(x, edge_index, W1l, b1l, W1r, W2l, b2l, W2r):
    N, D = x.shape
    E = edge_index.shape[1]
    H = W1l.shape[0]
    O = W2l.shape[0]
    F1 = H + 16
    Np = ((N + 8 * _NS - 1) // (8 * _NS)) * (8 * _NS)

    grain = _NW * _CH * 2
    Ep = ((E + grain - 1) // grain) * grain
    cpw2 = Ep // (_NS * _CH)
    k0_1, k0_2 = _split(cpw2, 205.3, 99.3), _split(cpw2, 94.7, 70.4)
    extra = max(k0_1, k0_2, cpw2 - min(k0_1, k0_2))
    n_pad = Ep - E + extra * _CH
    pad_dst = N + jnp.arange(n_pad, dtype=jnp.int32) % (Np - N)
    src = jnp.concatenate(
        [edge_index[0], jnp.zeros((n_pad,), jnp.int32)]).reshape(-1, _CH)
    dst = jnp.concatenate([edge_index[1], pad_dst]).reshape(-1, _CH)

    rpt = Np // _NS
    zeros1 = jnp.zeros((rpt, F1), jnp.float32)
    zeros2 = jnp.zeros((rpt, O), jnp.float32)

    ya1, r1 = _tc1(x, W1l, W1r)
    part1 = _make_sc_segsum(F1, Np, k0_1, cpw2 - k0_1)(ya1, src, dst, zeros1)
    y2, r2f, cnt = _tc2(part1, r1, b1l.reshape(1, H), W2l, W2r, b2l.reshape(1, O),
                        N, Np)
    part2 = _make_sc_segsum(O, Np, k0_2, cpw2 - k0_2)(y2, src, dst, zeros2)
    return _tc3(part2, cnt, r2f, N, Np)


# speedup vs baseline: 1.4486x; 1.4486x over previous
import functools

import jax
import jax.numpy as jnp
from jax import lax
from jax.experimental import pallas as pl
from jax.experimental.pallas import tpu as pltpu
from jax.experimental.pallas import tpu_sc as plsc

_NC = 2
_NS = 16
_NW = _NC * _NS
_CH = 128


def _make_sc_segsum(F, Np, k0, k1):
    rpt = Np // _NS
    kmax = max(k0, k1)
    mesh = plsc.VectorSubcoreMesh(core_axis_name="c", subcore_axis_name="s")

    @functools.partial(
        pl.kernel,
        out_type=jax.ShapeDtypeStruct((_NC * Np, F), jnp.float32),
        mesh=mesh,
        scratch_types=[
            pltpu.VMEM((kmax, _CH), jnp.int32),
            pltpu.VMEM((kmax, _CH), jnp.int32),
            pltpu.VMEM((_CH, F), jnp.float32),
            pltpu.VMEM((_CH, F), jnp.float32),
            pltpu.VMEM((rpt, F), jnp.float32),
            pltpu.VMEM_SHARED((Np, F), jnp.float32),
            pltpu.SemaphoreType.DMA,
            pltpu.SemaphoreType.DMA,
        ],
        compiler_params=pltpu.CompilerParams(use_tc_tiling_on_sc=False),
    )
    def k(table_hbm, src_hbm, dst_hbm, zeros_hbm, out_hbm,
          sidx_v, didx_v, rows0, rows1, buf_v, acc_sh, sem0, sem1):
        cid = lax.axis_index("c")
        sid = lax.axis_index("s")
        kc = jnp.where(cid == 0, k0, k1)
        row0 = cid * _NS * k0 + sid * kc

        with jax.named_scope("idx_preload"):
            pltpu.sync_copy(src_hbm.at[pl.ds(row0, kmax)], sidx_v)
            pltpu.sync_copy(dst_hbm.at[pl.ds(row0, kmax)], didx_v)

        with jax.named_scope("acc_init"):
            pltpu.sync_copy(zeros_hbm, buf_v)
            pltpu.sync_copy(buf_v, acc_sh.at[pl.ds(sid * rpt, rpt)])
            plsc.subcore_barrier()

        kc_half = kc // 2
        pltpu.async_copy(table_hbm.at[sidx_v.at[0]], rows0, sem0)

        def body(i):
            @pl.when(i < kc_half)
            def _():
                j0 = 2 * i
                pltpu.make_async_copy(table_hbm.at[sidx_v.at[j0]], rows0, sem0
                                      ).wait()
                pltpu.async_copy(table_hbm.at[sidx_v.at[j0 + 1]], rows1, sem1)
                pltpu.sync_copy(rows0, acc_sh.at[didx_v.at[j0]], add=True)
                pltpu.make_async_copy(table_hbm.at[sidx_v.at[j0 + 1]], rows1,
                                      sem1).wait()

                @pl.when(i < kc_half - 1)
                def _():
                    pltpu.async_copy(table_hbm.at[sidx_v.at[j0 + 2]], rows0,
                                     sem0)

                pltpu.sync_copy(rows1, acc_sh.at[didx_v.at[j0 + 1]], add=True)

        with jax.named_scope("edge_loop"):
            pl.loop(0, kmax // 2)(body)
            plsc.subcore_barrier()

        with jax.named_scope("writeout"):
            pltpu.sync_copy(acc_sh.at[pl.ds(sid * rpt, rpt)], buf_v)
            pltpu.sync_copy(buf_v,
                            out_hbm.at[pl.ds(cid * Np + sid * rpt, rpt)])

    return k


def _tc1(x, W1l, W1r):
    N, D = x.shape
    H = W1l.shape[0]
    F1 = H + 16

    def body(x_ref, wl_ref, wr_ref, ya_ref, r_ref):
        xb = x_ref[...]
        y = lax.dot_general(xb, wl_ref[...], (((1,), (1,)), ((), ())),
                            preferred_element_type=jnp.float32)
        ones_col = (lax.broadcasted_iota(jnp.int32, (N, 16), 1) == 0
                    ).astype(jnp.float32)
        ya_ref[...] = jnp.concatenate([y, ones_col], axis=1)
        r_ref[...] = lax.dot_general(xb, wr_ref[...], (((1,), (1,)), ((), ())),
                                     preferred_element_type=jnp.float32)

    return pl.pallas_call(
        body,
        out_shape=[jax.ShapeDtypeStruct((N, F1), jnp.float32),
                   jax.ShapeDtypeStruct((N, H), jnp.float32)],
    )(x, W1l, W1r)


def _tc2(part1, r1, b1l, W2l, W2r, b2l, N, Np):
    H = r1.shape[1]
    O = W2l.shape[0]

    def body(p_ref, r1_ref, b1l_ref, wl_ref, wr_ref, b2l_ref,
             y2_ref, r2_ref, cnt_ref):
        p = p_ref[:N] + p_ref[Np:Np + N]
        cnt = jnp.maximum(p[:, H:H + 1], 1.0)
        h = jnp.maximum(p[:, :H] / cnt + b1l_ref[...] + r1_ref[...], 0.0)
        y2_ref[...] = lax.dot_general(h, wl_ref[...], (((1,), (1,)), ((), ())),
                                      preferred_element_type=jnp.float32)
        r2_ref[...] = lax.dot_general(h, wr_ref[...], (((1,), (1,)), ((), ())),
                                      preferred_element_type=jnp.float32) \
            + b2l_ref[...]
        cnt_ref[...] = jnp.broadcast_to(cnt, (N, O))

    return pl.pallas_call(
        body,
        out_shape=[jax.ShapeDtypeStruct((N, O), jnp.float32),
                   jax.ShapeDtypeStruct((N, O), jnp.float32),
                   jax.ShapeDtypeStruct((N, O), jnp.float32)],
    )(part1, r1, b1l, W2l, W2r, b2l)


def _tc3(part2, cnt, r2f, N, Np):
    O = cnt.shape[1]

    def body(p_ref, cnt_ref, r2_ref, o_ref):
        o_ref[...] = (p_ref[:N] + p_ref[Np:Np + N]) / cnt_ref[...] + r2_ref[...]

    return pl.pallas_call(
        body,
        out_shape=jax.ShapeDtypeStruct((N, O), jnp.float32),
    )(part2, cnt, r2f)


def _split(total, t1, t0):
    k0 = int(round(total * t1 / (t0 + t1) / 2)) * 2
    return min(max(k0, 2), total - 2)


def kernel(x, edge_index, W1l, b1l, W1r, W2l, b2l, W2r):
    N, D = x.shape
    E = edge_index.shape[1]
    H = W1l.shape[0]
    O = W2l.shape[0]
    F1 = H + 16
    Np = ((N + 8 * _NS - 1) // (8 * _NS)) * (8 * _NS)

    grain = _NW * _CH * 2
    Ep = ((E + grain - 1) // grain) * grain
    cpw2 = Ep // (_NS * _CH)
    k0_1, k0_2 = cpw2 // 2, cpw2 // 2
    extra = max(k0_1, k0_2, cpw2 - min(k0_1, k0_2))
    n_pad = Ep - E + extra * _CH
    pad_idx = jnp.arange(n_pad, dtype=jnp.int32)
    pad_dst = N + pad_idx % (Np - N)
    pad_src = (pad_idx * 37) % N
    src = jnp.concatenate([edge_index[0], pad_src]).reshape(-1, _CH)
    dst = jnp.concatenate([edge_index[1], pad_dst]).reshape(-1, _CH)

    rpt = Np // _NS
    zeros1 = jnp.zeros((rpt, F1), jnp.float32)
    zeros2 = jnp.zeros((rpt, O), jnp.float32)

    ya1, r1 = _tc1(x, W1l, W1r)
    part1 = _make_sc_segsum(F1, Np, k0_1, cpw2 - k0_1)(ya1, src, dst, zeros1)
    y2, r2f, cnt = _tc2(part1, r1, b1l.reshape(1, H), W2l, W2r, b2l.reshape(1, O),
                        N, Np)
    part2 = _make_sc_segsum(O, Np, k0_2, cpw2 - k0_2)(y2, src, dst, zeros2)
    return _tc3(part2, cnt, r2f, N, Np)


# speedup vs baseline: 1.9868x; 1.3716x over previous
import functools

import jax
import jax.numpy as jnp
from jax import lax
from jax.experimental import pallas as pl
from jax.experimental.pallas import tpu as pltpu
from jax.experimental.pallas import tpu_sc as plsc

_NC = 2
_NS = 16
_NW = _NC * _NS
_CH = 128


def _make_sc_segsum(F, Np, n_chunks):
    rpt = Np // _NS
    base = n_chunks // _NW
    rem = n_chunks - base * _NW
    kpre = base + (1 if rem else 0)
    ni = base // 3
    leftover = base - ni * 3
    mesh = plsc.VectorSubcoreMesh(core_axis_name="c", subcore_axis_name="s")

    @functools.partial(
        pl.kernel,
        out_type=jax.ShapeDtypeStruct((_NC * Np, F), jnp.float32),
        mesh=mesh,
        scratch_types=[
            pltpu.VMEM((kpre, _CH), jnp.int32),
            pltpu.VMEM((kpre, _CH), jnp.int32),
            pltpu.VMEM((_CH, F), jnp.float32),
            pltpu.VMEM((_CH, F), jnp.float32),
            pltpu.VMEM((_CH, F), jnp.float32),
            pltpu.VMEM((rpt, F), jnp.float32),
            pltpu.VMEM_SHARED((Np, F), jnp.float32),
            pltpu.SemaphoreType.DMA,
            pltpu.SemaphoreType.DMA,
            pltpu.SemaphoreType.DMA,
            pltpu.SemaphoreType.DMA,
            pltpu.SemaphoreType.DMA,
            pltpu.SemaphoreType.DMA,
        ],
        compiler_params=pltpu.CompilerParams(use_tc_tiling_on_sc=False),
    )
    def k(table_hbm, src_hbm, dst_hbm, out_hbm,
          sidx_v, didx_v, rows0, rows1, rows2, buf_v, acc_sh,
          g0, g1, g2, s0, s1, s2):
        cid = lax.axis_index("c")
        sid = lax.axis_index("s")
        wid = sid * _NC + cid
        has_extra = wid >= _NW - rem
        row0 = wid * base + jnp.maximum(wid - (_NW - rem), 0)

        rows = (rows0, rows1, rows2)
        gsem = (g0, g1, g2)
        ssem = (s0, s1, s2)

        def gather_start(c, b):
            pltpu.async_copy(table_hbm.at[sidx_v.at[c]], rows[b], gsem[b])

        def gather_wait(c, b):
            pltpu.make_async_copy(table_hbm.at[sidx_v.at[c]], rows[b],
                                  gsem[b]).wait()

        def scatter_start(c, b):
            pltpu.async_copy(rows[b], acc_sh.at[didx_v.at[c]], ssem[b],
                             add=True)

        def scatter_wait(c, b):
            pltpu.make_async_copy(rows[b], acc_sh.at[didx_v.at[c]],
                                  ssem[b]).wait()

        with jax.named_scope("idx_preload"):
            pltpu.sync_copy(src_hbm.at[pl.ds(row0, kpre)], sidx_v)
            pltpu.sync_copy(dst_hbm.at[pl.ds(row0, kpre)], didx_v)

        with jax.named_scope("acc_init"):
            @pl.loop(0, rpt)
            def _(r):
                for c in range(F // 16):
                    buf_v[r, pl.ds(c * 16, 16)] = jnp.zeros((16,), jnp.float32)
            pltpu.sync_copy(buf_v, acc_sh.at[pl.ds(sid * rpt, rpt)])
            plsc.subcore_barrier()

        with jax.named_scope("edge_loop"):
            gather_start(0, 0)
            gather_start(1, 1)

            def body(i):
                c = 3 * i

                @pl.when(i > 0)
                def _():
                    scatter_wait(c - 1, 2)

                gather_start(c + 2, 2)
                gather_wait(c, 0)
                scatter_start(c, 0)
                gather_wait(c + 1, 1)
                scatter_start(c + 1, 1)
                scatter_wait(c, 0)

                @pl.when(i < ni - 1)
                def _():
                    gather_start(c + 3, 0)

                gather_wait(c + 2, 2)
                scatter_start(c + 2, 2)
                scatter_wait(c + 1, 1)

                @pl.when(i < ni - 1)
                def _():
                    gather_start(c + 4, 1)

            pl.loop(0, ni)(body)
            scatter_wait(3 * ni - 1, 2)

            for t in range(leftover):
                gather_start(3 * ni + t, 0)
                gather_wait(3 * ni + t, 0)
                scatter_start(3 * ni + t, 0)
                scatter_wait(3 * ni + t, 0)

            if rem:
                @pl.when(has_extra)
                def _():
                    gather_start(base, 0)
                    gather_wait(base, 0)
                    scatter_start(base, 0)
                    scatter_wait(base, 0)

            plsc.subcore_barrier()

        with jax.named_scope("writeout"):
            pltpu.sync_copy(acc_sh.at[pl.ds(sid * rpt, rpt)], buf_v)
            pltpu.sync_copy(buf_v,
                            out_hbm.at[pl.ds(cid * Np + sid * rpt, rpt)])

    return k


def _tc1(x, W1l, W1r):
    N, D = x.shape
    H = W1l.shape[0]
    F1 = H + 16

    def body(x_ref, wl_ref, wr_ref, ya_ref, r_ref):
        xb = x_ref[...]
        y = lax.dot_general(xb, wl_ref[...], (((1,), (1,)), ((), ())),
                            preferred_element_type=jnp.float32)
        ones_col = (lax.broadcasted_iota(jnp.int32, (N, 16), 1) == 0
                    ).astype(jnp.float32)
        ya_ref[...] = jnp.concatenate([y, ones_col], axis=1)
        r_ref[...] = lax.dot_general(xb, wr_ref[...], (((1,), (1,)), ((), ())),
                                     preferred_element_type=jnp.float32)

    return pl.pallas_call(
        body,
        out_shape=[jax.ShapeDtypeStruct((N, F1), jnp.float32),
                   jax.ShapeDtypeStruct((N, H), jnp.float32)],
    )(x, W1l, W1r)


def _tc2(part1, r1, b1l, W2l, W2r, b2l, N, Np):
    H = r1.shape[1]
    O = W2l.shape[0]

    def body(p_ref, r1_ref, b1l_ref, wl_ref, wr_ref, b2l_ref,
             y2_ref, r2_ref, cnt_ref):
        p = p_ref[:N] + p_ref[Np:Np + N]
        cnt = jnp.maximum(p[:, H:H + 1], 1.0)
        h = jnp.maximum(p[:, :H] / cnt + b1l_ref[...] + r1_ref[...], 0.0)
        y2_ref[...] = lax.dot_general(h, wl_ref[...], (((1,), (1,)), ((), ())),
                                      preferred_element_type=jnp.float32)
        r2_ref[...] = lax.dot_general(h, wr_ref[...], (((1,), (1,)), ((), ())),
                                      preferred_element_type=jnp.float32) \
            + b2l_ref[...]
        cnt_ref[...] = jnp.broadcast_to(cnt, (N, O))

    return pl.pallas_call(
        body,
        out_shape=[jax.ShapeDtypeStruct((N, O), jnp.float32),
                   jax.ShapeDtypeStruct((N, O), jnp.float32),
                   jax.ShapeDtypeStruct((N, O), jnp.float32)],
    )(part1, r1, b1l, W2l, W2r, b2l)


def _tc3(part2, cnt, r2f, N, Np):
    O = cnt.shape[1]

    def body(p_ref, cnt_ref, r2_ref, o_ref):
        o_ref[...] = (p_ref[:N] + p_ref[Np:Np + N]) / cnt_ref[...] + r2_ref[...]

    return pl.pallas_call(
        body,
        out_shape=jax.ShapeDtypeStruct((N, O), jnp.float32),
    )(part2, cnt, r2f)


def kernel(x, edge_index, W1l, b1l, W1r, W2l, b2l, W2r):
    N, D = x.shape
    E = edge_index.shape[1]
    H = W1l.shape[0]
    O = W2l.shape[0]
    F1 = H + 16
    Np = ((N + 8 * _NS - 1) // (8 * _NS)) * (8 * _NS)

    n_chunks = E // _CH
    src = edge_index[0].reshape(n_chunks, _CH)
    dst = edge_index[1].reshape(n_chunks, _CH)

    ya1, r1 = _tc1(x, W1l, W1r)
    part1 = _make_sc_segsum(F1, Np, n_chunks)(ya1, src, dst)
    y2, r2f, cnt = _tc2(part1, r1, b1l.reshape(1, H), W2l, W2r, b2l.reshape(1, O),
                        N, Np)
    part2 = _make_sc_segsum(O, Np, n_chunks)(y2, src, dst)
    return _tc3(part2, cnt, r2f, N, Np)


# speedup vs baseline: 2.0110x; 1.0122x over previous
import functools

import jax
import jax.numpy as jnp
from jax import lax
from jax.experimental import pallas as pl
from jax.experimental.pallas import tpu as pltpu
from jax.experimental.pallas import tpu_sc as plsc

_NC = 2
_NS = 16
_NW = _NC * _NS
_CH = 128


def _make_sc_segsum(F, Np, n_chunks, with_counts=False):
    rpt = Np // _NS
    base = n_chunks // _NW
    rem = n_chunks - base * _NW
    kpre = base + (1 if rem else 0)
    ni = base // 3
    leftover = base - ni * 3
    CR = ((Np // 16 + _CH - 1) // _CH) * _CH
    mesh = plsc.VectorSubcoreMesh(core_axis_name="c", subcore_axis_name="s")

    cnt_out = ([jax.ShapeDtypeStruct((_NC * CR, 16), jnp.float32)]
               if with_counts else [])
    cnt_scratch = ([pltpu.VMEM((CR, 16), jnp.float32),
                    pltpu.VMEM((CR // _CH, _CH), jnp.int32),
                    pltpu.VMEM((CR // _NS, 16), jnp.float32),
                    pltpu.VMEM_SHARED((CR, 16), jnp.float32)]
                   if with_counts else [])

    @functools.partial(
        pl.kernel,
        out_type=[jax.ShapeDtypeStruct((_NC * Np, F), jnp.float32)] + cnt_out,
        mesh=mesh,
        scratch_types=[
            pltpu.VMEM((kpre, _CH), jnp.int32),
            pltpu.VMEM((kpre, _CH), jnp.int32),
            pltpu.VMEM((_CH, F), jnp.float32),
            pltpu.VMEM((_CH, F), jnp.float32),
            pltpu.VMEM((_CH, F), jnp.float32),
            pltpu.VMEM((rpt, F), jnp.float32),
            pltpu.VMEM_SHARED((Np, F), jnp.float32),
        ] + cnt_scratch + [
            pltpu.SemaphoreType.DMA,
            pltpu.SemaphoreType.DMA,
            pltpu.SemaphoreType.DMA,
            pltpu.SemaphoreType.DMA,
            pltpu.SemaphoreType.DMA,
            pltpu.SemaphoreType.DMA,
        ],
        compiler_params=pltpu.CompilerParams(use_tc_tiling_on_sc=False, needs_layout_passes=False),
    )
    def k(table_hbm, src_hbm, dst_hbm, *rest):
        if with_counts:
            (out_hbm, cnt_hbm, sidx_v, didx_v, rows0, rows1, rows2, buf_v,
             acc_sh, hist_v, iota_v, cbuf_v, accC_sh,
             g0, g1, g2, s0, s1, s2) = rest
        else:
            (out_hbm, sidx_v, didx_v, rows0, rows1, rows2, buf_v, acc_sh,
             g0, g1, g2, s0, s1, s2) = rest
        cid = lax.axis_index("c")
        sid = lax.axis_index("s")
        wid = sid * _NC + cid
        has_extra = wid >= _NW - rem
        row0 = wid * base + jnp.maximum(wid - (_NW - rem), 0)

        rows = (rows0, rows1, rows2)
        gsem = (g0, g1, g2)
        ssem = (s0, s1, s2)

        def gather_start(c, b):
            pltpu.async_copy(table_hbm.at[sidx_v.at[c]], rows[b], gsem[b])

        def gather_wait(c, b):
            pltpu.make_async_copy(table_hbm.at[sidx_v.at[c]], rows[b],
                                  gsem[b]).wait()

        def scatter_start(c, b):
            pltpu.async_copy(rows[b], acc_sh.at[didx_v.at[c]], ssem[b],
                             add=True)

        def scatter_wait(c, b):
            pltpu.make_async_copy(rows[b], acc_sh.at[didx_v.at[c]],
                                  ssem[b]).wait()

        with jax.named_scope("idx_preload"):
            pltpu.sync_copy(src_hbm.at[pl.ds(row0, kpre)], sidx_v)
            pltpu.sync_copy(dst_hbm.at[pl.ds(row0, kpre)], didx_v)

        with jax.named_scope("acc_init"):
            @pl.loop(0, rpt)
            def _(r):
                for c in range(F // 16):
                    buf_v[r, pl.ds(c * 16, 16)] = jnp.zeros((16,), jnp.float32)
            pltpu.sync_copy(buf_v, acc_sh.at[pl.ds(sid * rpt, rpt)])
            if with_counts:
                crt = CR // _NS
                @pl.loop(0, CR)
                def _(r):
                    hist_v[r, :] = jnp.zeros((16,), jnp.float32)
                pltpu.sync_copy(hist_v.at[pl.ds(0, crt)],
                                accC_sh.at[pl.ds(sid * crt, crt)])
                for j in range(CR // _CH):
                    for g in range(_CH // 16):
                        iota_v[j, pl.ds(g * 16, 16)] = (
                            lax.iota(jnp.int32, 16) + (j * _CH + g * 16))
            plsc.subcore_barrier()

        if with_counts:
            with jax.named_scope("histogram"):
                ones16 = jnp.ones((16,), jnp.float32)

                def hchunk(c):
                    for g in range(_CH // 16):
                        d = didx_v[c, pl.ds(g * 16, 16)]
                        plsc.addupdate_scatter(
                            hist_v, [d >> 4, d & 15], ones16)

                pl.loop(0, base)(hchunk)
                if rem:
                    @pl.when(has_extra)
                    def _():
                        hchunk(base)
                for j in range(CR // _CH):
                    pltpu.sync_copy(hist_v.at[pl.ds(j * _CH, _CH)],
                                    accC_sh.at[iota_v.at[j]], add=True)

        with jax.named_scope("edge_loop"):
            gather_start(0, 0)
            gather_start(1, 1)

            def body(i):
                c = 3 * i

                @pl.when(i > 0)
                def _():
                    scatter_wait(c - 1, 2)

                gather_start(c + 2, 2)
                gather_wait(c, 0)
                scatter_start(c, 0)
                gather_wait(c + 1, 1)
                scatter_start(c + 1, 1)
                scatter_wait(c, 0)

                @pl.when(i < ni - 1)
                def _():
                    gather_start(c + 3, 0)

                gather_wait(c + 2, 2)
                scatter_start(c + 2, 2)
                scatter_wait(c + 1, 1)

                @pl.when(i < ni - 1)
                def _():
                    gather_start(c + 4, 1)

            pl.loop(0, ni)(body)
            scatter_wait(3 * ni - 1, 2)

            for t in range(leftover):
                gather_start(3 * ni + t, 0)
                gather_wait(3 * ni + t, 0)
                scatter_start(3 * ni + t, 0)
                scatter_wait(3 * ni + t, 0)

            if rem:
                @pl.when(has_extra)
                def _():
                    gather_start(base, 0)
                    gather_wait(base, 0)
                    scatter_start(base, 0)
                    scatter_wait(base, 0)

            plsc.subcore_barrier()

        with jax.named_scope("writeout"):
            pltpu.sync_copy(acc_sh.at[pl.ds(sid * rpt, rpt)], buf_v)
            pltpu.sync_copy(buf_v,
                            out_hbm.at[pl.ds(cid * Np + sid * rpt, rpt)])
            if with_counts:
                crt = CR // _NS
                pltpu.sync_copy(accC_sh.at[pl.ds(sid * crt, crt)], cbuf_v)
                pltpu.sync_copy(cbuf_v,
                                cnt_hbm.at[pl.ds(cid * CR + sid * crt, crt)])

    return k


def _tc1(x, W1l, W1r):
    N, D = x.shape
    H = W1l.shape[0]

    def body(x_ref, wl_ref, wr_ref, y_ref, r_ref):
        xb = x_ref[...]
        y_ref[...] = lax.dot_general(xb, wl_ref[...], (((1,), (1,)), ((), ())),
                                     preferred_element_type=jnp.float32)
        r_ref[...] = lax.dot_general(xb, wr_ref[...], (((1,), (1,)), ((), ())),
                                     preferred_element_type=jnp.float32)

    return pl.pallas_call(
        body,
        out_shape=[jax.ShapeDtypeStruct((N, H), jnp.float32),
                   jax.ShapeDtypeStruct((N, H), jnp.float32)],
    )(x, W1l, W1r)


def _tc2(part1, cnt_col, r1, b1l, W2l, W2r, b2l, N, Np):
    H = r1.shape[1]
    O = W2l.shape[0]

    def body(p_ref, c_ref, r1_ref, b1l_ref, wl_ref, wr_ref, b2l_ref,
             y2_ref, r2_ref, icnt_ref):
        p = p_ref[:N] + p_ref[Np:Np + N]
        icnt = 1.0 / jnp.maximum(c_ref[...], 1.0)
        h = jnp.maximum(p * icnt + b1l_ref[...] + r1_ref[...], 0.0)
        y2_ref[...] = lax.dot_general(h, wl_ref[...], (((1,), (1,)), ((), ())),
                                      preferred_element_type=jnp.float32)
        r2_ref[...] = lax.dot_general(h, wr_ref[...], (((1,), (1,)), ((), ())),
                                      preferred_element_type=jnp.float32) \
            + b2l_ref[...]
        icnt_ref[...] = jnp.broadcast_to(icnt, (N, O))

    return pl.pallas_call(
        body,
        out_shape=[jax.ShapeDtypeStruct((N, O), jnp.float32),
                   jax.ShapeDtypeStruct((N, O), jnp.float32),
                   jax.ShapeDtypeStruct((N, O), jnp.float32)],
    )(part1, cnt_col, r1, b1l, W2l, W2r, b2l)


def _tc3(part2, icnt, r2f, N, Np):
    O = icnt.shape[1]

    def body(p_ref, cnt_ref, r2_ref, o_ref):
        o_ref[...] = (p_ref[:N] + p_ref[Np:Np + N]) * cnt_ref[...] + r2_ref[...]

    return pl.pallas_call(
        body,
        out_shape=jax.ShapeDtypeStruct((N, O), jnp.float32),
    )(part2, icnt, r2f)


def kernel(x, edge_index, W1l, b1l, W1r, W2l, b2l, W2r):
    N, D = x.shape
    E = edge_index.shape[1]
    H = W1l.shape[0]
    O = W2l.shape[0]
    F1 = H + 16
    Np = ((N + 8 * _NS - 1) // (8 * _NS)) * (8 * _NS)

    n_chunks = E // _CH
    src = edge_index[0].reshape(n_chunks, _CH)
    dst = edge_index[1].reshape(n_chunks, _CH)

    y1, r1 = _tc1(x, W1l, W1r)
    part1, cntp = _make_sc_segsum(H, Np, n_chunks, with_counts=True)(
        y1, src, dst)
    CR = cntp.shape[0] // _NC
    cnt_col = (cntp[:CR] + cntp[CR:]).reshape(-1)[:N].reshape(N, 1)
    y2, r2f, icnt = _tc2(part1, cnt_col, r1, b1l.reshape(1, H),
                         W2l, W2r, b2l.reshape(1, O), N, Np)
    part2, = _make_sc_segsum(O, Np, n_chunks)(y2, src, dst)
    return _tc3(part2, icnt, r2f, N, Np)


# speedup vs baseline: 2.0626x; 1.0257x over previous
import functools

import jax
import jax.numpy as jnp
from jax import lax
from jax.experimental import pallas as pl
from jax.experimental.pallas import tpu as pltpu
from jax.experimental.pallas import tpu_sc as plsc

_NC = 2
_NS = 16
_NW = _NC * _NS
_CH = 128


def _make_sc_segsum(F, Np, n_chunks, with_counts=False):
    rpt = Np // _NS
    base = n_chunks // _NW
    rem = n_chunks - base * _NW
    kpre = base + (1 if rem else 0)
    ni = base // 3
    leftover = base - ni * 3
    CR = ((Np // 16 + _CH - 1) // _CH) * _CH
    mesh = plsc.VectorSubcoreMesh(core_axis_name="c", subcore_axis_name="s")

    cnt_out = ([jax.ShapeDtypeStruct((_NC * CR, 16), jnp.float32)]
               if with_counts else [])
    cnt_scratch = ([pltpu.VMEM((CR, 16), jnp.float32),
                    pltpu.VMEM((CR // _CH, _CH), jnp.int32),
                    pltpu.VMEM((CR // _NS, 16), jnp.float32),
                    pltpu.VMEM_SHARED((CR, 16), jnp.float32)]
                   if with_counts else [])

    @functools.partial(
        pl.kernel,
        out_type=[jax.ShapeDtypeStruct((_NC * Np, F), jnp.float32)] + cnt_out,
        mesh=mesh,
        scratch_types=[
            pltpu.VMEM((kpre, _CH), jnp.int32),
            pltpu.VMEM((kpre, _CH), jnp.int32),
            pltpu.VMEM((_CH, F), jnp.float32),
            pltpu.VMEM((_CH, F), jnp.float32),
            pltpu.VMEM((_CH, F), jnp.float32),
            pltpu.VMEM((rpt, F), jnp.float32),
            pltpu.VMEM_SHARED((Np, F), jnp.float32),
        ] + cnt_scratch + [
            pltpu.SemaphoreType.DMA,
            pltpu.SemaphoreType.DMA,
            pltpu.SemaphoreType.DMA,
            pltpu.SemaphoreType.DMA,
            pltpu.SemaphoreType.DMA,
            pltpu.SemaphoreType.DMA,
        ],
        compiler_params=pltpu.CompilerParams(use_tc_tiling_on_sc=False, needs_layout_passes=False),
    )
    def k(table_hbm, src_hbm, dst_hbm, *rest):
        if with_counts:
            (out_hbm, cnt_hbm, sidx_v, didx_v, rows0, rows1, rows2, buf_v,
             acc_sh, hist_v, iota_v, cbuf_v, accC_sh,
             g0, g1, g2, s0, s1, s2) = rest
        else:
            (out_hbm, sidx_v, didx_v, rows0, rows1, rows2, buf_v, acc_sh,
             g0, g1, g2, s0, s1, s2) = rest
        cid = lax.axis_index("c")
        sid = lax.axis_index("s")
        wid = sid * _NC + cid
        has_extra = wid >= _NW - rem
        row0 = wid * base + jnp.maximum(wid - (_NW - rem), 0)

        rows = (rows0, rows1, rows2)
        gsem = (g0, g1, g2)
        ssem = (s0, s1, s2)

        def gather_start(c, b):
            pltpu.async_copy(table_hbm.at[sidx_v.at[c]], rows[b], gsem[b])

        def gather_wait(c, b):
            pltpu.make_async_copy(table_hbm.at[sidx_v.at[c]], rows[b],
                                  gsem[b]).wait()

        def scatter_start(c, b):
            pltpu.async_copy(rows[b], acc_sh.at[didx_v.at[c]], ssem[b],
                             add=True)

        def scatter_wait(c, b):
            pltpu.make_async_copy(rows[b], acc_sh.at[didx_v.at[c]],
                                  ssem[b]).wait()

        with jax.named_scope("idx_preload"):
            pltpu.sync_copy(src_hbm.at[pl.ds(row0, kpre)], sidx_v)
            pltpu.sync_copy(dst_hbm.at[pl.ds(row0, kpre)], didx_v)

        with jax.named_scope("acc_init"):
            @pl.loop(0, rpt)
            def _(r):
                for c in range(F // 16):
                    buf_v[r, pl.ds(c * 16, 16)] = jnp.zeros((16,), jnp.float32)
            pltpu.sync_copy(buf_v, acc_sh.at[pl.ds(sid * rpt, rpt)])
            if with_counts:
                crt = CR // _NS
                @pl.loop(0, CR)
                def _(r):
                    hist_v[r, :] = jnp.zeros((16,), jnp.float32)
                pltpu.sync_copy(hist_v.at[pl.ds(0, crt)],
                                accC_sh.at[pl.ds(sid * crt, crt)])
                for j in range(CR // _CH):
                    for g in range(_CH // 16):
                        iota_v[j, pl.ds(g * 16, 16)] = (
                            lax.iota(jnp.int32, 16) + (j * _CH + g * 16))
            plsc.subcore_barrier()

        ones16 = jnp.ones((16,), jnp.float32)

        def hchunk(c):
            if with_counts:
                for g in range(_CH // 16):
                    d = didx_v[c, pl.ds(g * 16, 16)]
                    plsc.addupdate_scatter(hist_v, [d >> 4, d & 15], ones16)

        with jax.named_scope("edge_loop"):
            gather_start(0, 0)
            gather_start(1, 1)

            def body(i):
                c = 3 * i

                @pl.when(i > 0)
                def _():
                    scatter_wait(c - 1, 2)

                gather_start(c + 2, 2)
                gather_wait(c, 0)
                scatter_start(c, 0)
                hchunk(c)
                gather_wait(c + 1, 1)
                scatter_start(c + 1, 1)
                hchunk(c + 1)
                scatter_wait(c, 0)

                @pl.when(i < ni - 1)
                def _():
                    gather_start(c + 3, 0)

                gather_wait(c + 2, 2)
                scatter_start(c + 2, 2)
                hchunk(c + 2)
                scatter_wait(c + 1, 1)

                @pl.when(i < ni - 1)
                def _():
                    gather_start(c + 4, 1)

            pl.loop(0, ni)(body)
            scatter_wait(3 * ni - 1, 2)

            for t in range(leftover):
                gather_start(3 * ni + t, 0)
                gather_wait(3 * ni + t, 0)
                scatter_start(3 * ni + t, 0)
                hchunk(3 * ni + t)
                scatter_wait(3 * ni + t, 0)

            if rem:
                @pl.when(has_extra)
                def _():
                    gather_start(base, 0)
                    gather_wait(base, 0)
                    scatter_start(base, 0)
                    hchunk(base)
                    scatter_wait(base, 0)

            if with_counts:
                for j in range(CR // _CH):
                    pltpu.sync_copy(hist_v.at[pl.ds(j * _CH, _CH)],
                                    accC_sh.at[iota_v.at[j]], add=True)

            plsc.subcore_barrier()

        with jax.named_scope("writeout"):
            pltpu.sync_copy(acc_sh.at[pl.ds(sid * rpt, rpt)], buf_v)
            pltpu.sync_copy(buf_v,
                            out_hbm.at[pl.ds(cid * Np + sid * rpt, rpt)])
            if with_counts:
                crt = CR // _NS
                pltpu.sync_copy(accC_sh.at[pl.ds(sid * crt, crt)], cbuf_v)
                pltpu.sync_copy(cbuf_v,
                                cnt_hbm.at[pl.ds(cid * CR + sid * crt, crt)])

    return k


def _tc1(x, W1l, W1r):
    N, D = x.shape
    H = W1l.shape[0]

    def body(x_ref, wl_ref, wr_ref, y_ref, r_ref):
        xb = x_ref[...]
        y_ref[...] = lax.dot_general(xb, wl_ref[...], (((1,), (1,)), ((), ())),
                                     preferred_element_type=jnp.float32)
        r_ref[...] = lax.dot_general(xb, wr_ref[...], (((1,), (1,)), ((), ())),
                                     preferred_element_type=jnp.float32)

    return pl.pallas_call(
        body,
        out_shape=[jax.ShapeDtypeStruct((N, H), jnp.float32),
                   jax.ShapeDtypeStruct((N, H), jnp.float32)],
    )(x, W1l, W1r)


def _tc2(part1, cnt_col, r1, b1l, W2l, W2r, b2l, N, Np):
    H = r1.shape[1]
    O = W2l.shape[0]

    def body(p_ref, c_ref, r1_ref, b1l_ref, wl_ref, wr_ref, b2l_ref,
             y2_ref, r2_ref, icnt_ref):
        p = p_ref[:N] + p_ref[Np:Np + N]
        icnt = 1.0 / jnp.maximum(c_ref[...], 1.0)
        h = jnp.maximum(p * icnt + b1l_ref[...] + r1_ref[...], 0.0)
        y2_ref[...] = lax.dot_general(h, wl_ref[...], (((1,), (1,)), ((), ())),
                                      preferred_element_type=jnp.float32)
        r2_ref[...] = lax.dot_general(h, wr_ref[...], (((1,), (1,)), ((), ())),
                                      preferred_element_type=jnp.float32) \
            + b2l_ref[...]
        icnt_ref[...] = jnp.broadcast_to(icnt, (N, O))

    return pl.pallas_call(
        body,
        out_shape=[jax.ShapeDtypeStruct((N, O), jnp.float32),
                   jax.ShapeDtypeStruct((N, O), jnp.float32),
                   jax.ShapeDtypeStruct((N, O), jnp.float32)],
    )(part1, cnt_col, r1, b1l, W2l, W2r, b2l)


def _tc3(part2, icnt, r2f, N, Np):
    O = icnt.shape[1]

    def body(p_ref, cnt_ref, r2_ref, o_ref):
        o_ref[...] = (p_ref[:N] + p_ref[Np:Np + N]) * cnt_ref[...] + r2_ref[...]

    return pl.pallas_call(
        body,
        out_shape=jax.ShapeDtypeStruct((N, O), jnp.float32),
    )(part2, icnt, r2f)


def kernel(x, edge_index, W1l, b1l, W1r, W2l, b2l, W2r):
    N, D = x.shape
    E = edge_index.shape[1]
    H = W1l.shape[0]
    O = W2l.shape[0]
    F1 = H + 16
    Np = ((N + 8 * _NS - 1) // (8 * _NS)) * (8 * _NS)

    n_chunks = E // _CH
    src = edge_index[0].reshape(n_chunks, _CH)
    dst = edge_index[1].reshape(n_chunks, _CH)

    y1, r1 = _tc1(x, W1l, W1r)
    part1, cntp = _make_sc_segsum(H, Np, n_chunks, with_counts=True)(
        y1, src, dst)
    CR = cntp.shape[0] // _NC
    cnt_col = (cntp[:CR] + cntp[CR:]).reshape(-1)[:N].reshape(N, 1)
    y2, r2f, icnt = _tc2(part1, cnt_col, r1, b1l.reshape(1, H),
                         W2l, W2r, b2l.reshape(1, O), N, Np)
    part2, = _make_sc_segsum(O, Np, n_chunks)(y2, src, dst)
    return _tc3(part2, icnt, r2f, N, Np)
